# baseline (speedup 1.0000x reference)
import jax
import jax.numpy as jnp
from jax.experimental import pallas as pl


_BS = 512


def _add_body(x_ref, pos_ref, o_ref):
    o_ref[...] = x_ref[...] + pos_ref[...][:, None, :]


def _part(x, pos_table, row0, rows):
    s, batch, d = x.shape
    return pl.pallas_call(
        _add_body,
        grid=(rows // _BS,),
        in_specs=[
            pl.BlockSpec((_BS, batch, d), lambda i: (i + row0 // _BS, 0, 0)),
            pl.BlockSpec((_BS, d), lambda i: (i + row0 // _BS, 0)),
        ],
        out_specs=pl.BlockSpec((_BS, batch, d), lambda i: (i, 0, 0)),
        out_shape=jax.ShapeDtypeStruct((rows, batch, d), x.dtype),
    )(x, pos_table)


def kernel(x, pos_table):
    s = x.shape[0]
    half = s // 2
    lo = _part(x, pos_table, 0, half)
    hi = _part(x, pos_table, half, s - half)
    return jnp.concatenate([lo, hi], axis=0)
